# baseline (speedup 1.0000x reference)
import functools

import jax
import jax.numpy as jnp
from jax import lax
from jax.experimental import pallas as pl
from jax.experimental.pallas import tpu as pltpu
from jax.experimental.pallas import tpu_sc as plsc

B = 16384
D = 32
LANES = 128
ROWS = 1000000

_info = plsc.get_sparse_core_info()
NC = _info.num_cores
NS = _info.num_subcores
L = _info.num_lanes
NW = NC * NS
BPW = B // NW

WPT = 245
SPAN = 5 * LANES
NCHK = 49
MAXOFF = 999424
DUMP = B
WLCAP = 1056
STCAP = 272
SCATB = L * LANES * 4


def _sc_extract(dflat, ut, it):
    mesh = plsc.VectorSubcoreMesh(core_axis_name="c", subcore_axis_name="s")
    stage_t = jax.ShapeDtypeStruct((B + L, LANES), jnp.float32)

    @functools.partial(
        pl.kernel,
        mesh=mesh,
        compiler_params=pltpu.CompilerParams(
            needs_layout_passes=False, use_tc_tiling_on_sc=True),
        out_type=(stage_t, stage_t),
        scratch_types=[
            pltpu.VMEM((2 * B,), jnp.int32),
            pltpu.VMEM((B,), jnp.int32),
            pltpu.VMEM((B,), jnp.int32),
            pltpu.VMEM((WLCAP,), jnp.int32),
            pltpu.VMEM((WLCAP,), jnp.int32),
            pltpu.VMEM((STCAP,), jnp.int32),
            pltpu.VMEM((STCAP,), jnp.int32),
            pltpu.VMEM((2, D, SPAN), jnp.float32),
            pltpu.VMEM((4, L, LANES), jnp.float32),
            pltpu.VMEM((4, L), jnp.int32),
            pltpu.SemaphoreType.DMA,
            pltpu.SemaphoreType.DMA,
            pltpu.SemaphoreType.DMA,
        ],
    )
    def k(d_hbm, ut_hbm, it_hbm, u_out, v_out,
          data_vm, users, items, wl_i, wl_b, st_l, st_b,
          ring, st_src, b2d, semf0, semf1, ssem):
        wid = lax.axis_index("s") * NC + lax.axis_index("c")
        lo = wid * (WPT * LANES)

        pltpu.sync_copy(d_hbm, data_vm)

        iota = lax.iota(jnp.int32, L)

        def deint(g, carry):
            p0 = pl.multiple_of(g * L, L)
            two = 2 * p0 + 2 * iota
            users[pl.ds(p0, L)] = plsc.load_gather(data_vm, [two])
            items[pl.ds(p0, L)] = plsc.load_gather(data_vm, [two + 1])
            return carry

        lax.fori_loop(0, B // L, deint, 0)

        fsems = (semf0, semf1)

        for lst, tbl_hbm, stg in ((users, ut_hbm, u_out),
                                  (items, it_hbm, v_out)):
            def wbuild(g, cnt):
                vec = lst[pl.ds(pl.multiple_of(g * L, L), L)]
                m = (vec >= lo) & (vec < lo + WPT * LANES)
                plsc.store_compressed(wl_i.at[pl.ds(cnt, L)], vec, mask=m)
                plsc.store_compressed(
                    wl_b.at[pl.ds(cnt, L)], g * L + iota, mask=m)
                return cnt + jnp.max(plsc.all_reduce_population_count(m))

            cnt = lax.fori_loop(0, B // L, wbuild, jnp.int32(0))
            wl_i[pl.ds(cnt, L)] = jnp.full((L,), -1, jnp.int32)
            wl_b[pl.ds(cnt, L)] = jnp.full((L,), DUMP, jnp.int32)
            nwl = (cnt + L - 1) // L

            def fetch(c, slot, sem):
                base = lo + c * SPAN
                off = pl.multiple_of(
                    jnp.minimum(base, MAXOFF), LANES)
                pltpu.async_copy(
                    tbl_hbm.at[:, pl.ds(off, SPAN)], ring.at[slot], sem)

            def drainf(slot, sem):
                pltpu.make_async_copy(
                    tbl_hbm.at[:, pl.ds(0, SPAN)], ring.at[slot],
                    sem).wait()

            def process(c, slot, rc):
                base = lo + c * SPAN
                off = jnp.minimum(base, MAXOFF)

                def scan(j, cnt2):
                    j0 = pl.multiple_of(j * L, L)
                    wi = wl_i[pl.ds(j0, L)]
                    wb = wl_b[pl.ds(j0, L)]
                    m2 = (wi >= base) & (wi < base + SPAN)
                    plsc.store_compressed(
                        st_l.at[pl.ds(cnt2, L)], wi - off, mask=m2)
                    plsc.store_compressed(
                        st_b.at[pl.ds(cnt2, L)], wb, mask=m2)
                    return cnt2 + jnp.max(
                        plsc.all_reduce_population_count(m2))

                cnt2 = lax.fori_loop(0, nwl, scan, jnp.int32(0))
                st_l[pl.ds(cnt2, L)] = jnp.zeros((L,), jnp.int32)
                st_b[pl.ds(cnt2, L)] = jnp.full((L,), DUMP, jnp.int32)

                def group(gi, rc_in):
                    g0 = pl.multiple_of(gi * L, L)
                    lane_v = st_l[pl.ds(g0, L)]
                    b_v = st_b[pl.ds(g0, L)]
                    slot4 = lax.rem(rc_in, 4)

                    @pl.when(rc_in >= 4)
                    def _():
                        pltpu.make_async_copy(
                            st_src.at[0], stg.at[b2d.at[0]], ssem).wait()

                    b2d[slot4, pl.ds(0, L)] = b_v
                    s4v = jnp.full((L,), slot4, jnp.int32)
                    for j in range(D):
                        jv = jnp.full((L,), j, jnp.int32)
                        vals = plsc.load_gather(
                            ring.at[slot], [jv, lane_v])
                        plsc.store_scatter(st_src, [s4v, iota, jv], vals)
                    pltpu.async_copy(
                        st_src.at[slot4], stg.at[b2d.at[slot4]], ssem)
                    return rc_in + 1

                return lax.fori_loop(0, (cnt2 + L - 1) // L, group, rc)

            fetch(0, 0, semf0)

            def blk(cb, rc):
                c0 = 2 * cb
                fetch(c0 + 1, 1, semf1)
                drainf(0, semf0)
                rc = process(c0, 0, rc)
                fetch(c0 + 2, 0, semf0)
                drainf(1, semf1)
                rc = process(c0 + 1, 1, rc)
                return rc

            rc = lax.fori_loop(0, (NCHK - 1) // 2, blk, jnp.int32(0))
            drainf(0, semf0)
            rc = process(NCHK - 1, 0, rc)

            def sdrain(i, carry):
                pltpu.make_async_copy(
                    st_src.at[0], stg.at[b2d.at[0]], ssem).wait()
                return carry

            lax.fori_loop(0, jnp.minimum(rc, 4), sdrain, 0)

    return k(dflat, ut, it)


def _sc_join(u_stage, v_stage):
    mesh = plsc.VectorSubcoreMesh(core_axis_name="c", subcore_axis_name="s")
    SUB = 128

    @functools.partial(
        pl.kernel,
        mesh=mesh,
        compiler_params=pltpu.CompilerParams(
            needs_layout_passes=False, use_tc_tiling_on_sc=True),
        out_type=jax.ShapeDtypeStruct((B,), jnp.float32),
        scratch_types=[
            pltpu.VMEM((SUB, LANES), jnp.float32),
            pltpu.VMEM((SUB, LANES), jnp.float32),
            pltpu.VMEM((BPW,), jnp.float32),
        ],
    )
    def k(u_hbm, v_hbm, out_hbm, ub, vb, out_v):
        wid = lax.axis_index("s") * NC + lax.axis_index("c")
        iota = lax.iota(jnp.int32, L)
        for sub in range(BPW // SUB):
            row0 = pl.multiple_of(wid * BPW + sub * SUB, 8)
            pltpu.sync_copy(u_hbm.at[pl.ds(row0, SUB)], ub)
            pltpu.sync_copy(v_hbm.at[pl.ds(row0, SUB)], vb)

            def dot16(g, carry, sub=sub):
                rl = g * L + iota
                acc = jnp.zeros((L,), jnp.float32)
                for j in range(D):
                    jv = jnp.full((L,), j, jnp.int32)
                    acc = acc + (plsc.load_gather(ub, [rl, jv])
                                 * plsc.load_gather(vb, [rl, jv]))
                o0 = pl.multiple_of(sub * SUB + g * L, L)
                out_v[pl.ds(o0, L)] = acc
                return carry

            lax.fori_loop(0, SUB // L, dot16, 0)

        base = pl.multiple_of(wid * BPW, BPW)
        pltpu.sync_copy(out_v, out_hbm.at[pl.ds(base, BPW)])

    return k(u_stage, v_stage)


def kernel(data, user_factors, item_factors):
    dflat = data.astype(jnp.int32).reshape(-1)
    u_stage, v_stage = _sc_extract(
        dflat, user_factors.T, item_factors.T)
    return _sc_join(u_stage, v_stage)
